# Initial kernel scaffold; baseline (speedup 1.0000x reference)
import jax
import jax.numpy as jnp
from jax.experimental import pallas as pl


def kernel(x, age, emb_table, fc_w, fc_b, hid_w, hid_b):
    raise NotImplementedError("write your pallas kernel here")


# speedup vs baseline: 1.8683x; 1.8683x over previous
import functools

import jax
import jax.numpy as jnp
from jax import lax
from jax.experimental import pallas as pl
from jax.experimental.pallas import tpu as pltpu
from jax.experimental.pallas import tpu_sc as plsc

VOCAB = 1000000
EMB = 32
HIDDEN = 50
OUT = 100
SEQ = 200
BATCH = 4096

NC = 2
NS = 16
NW = NC * NS
BPW = BATCH // NW
NCHUNK = 2
CHUNK = SEQ // NCHUNK
UNROLL = 8


def _pooled_sum_sc(xt3, emb_table):
    mesh = plsc.VectorSubcoreMesh(core_axis_name="c", subcore_axis_name="s")

    @functools.partial(
        pl.kernel,
        mesh=mesh,
        compiler_params=pltpu.CompilerParams(use_tc_tiling_on_sc=False),
        out_type=jax.ShapeDtypeStruct((BATCH, EMB), jnp.float32),
        scratch_types=[
            pltpu.VMEM((BPW, NCHUNK, CHUNK), jnp.int32),
            pltpu.VMEM((SEQ, EMB), jnp.float32),
            pltpu.VMEM((SEQ, EMB), jnp.float32),
            pltpu.VMEM((BPW, EMB), jnp.float32),
            pltpu.SemaphoreType.DMA,
            pltpu.SemaphoreType.DMA,
        ],
    )
    def pool_k(x_hbm, tab_hbm, out_hbm, idx_v, rows_a, rows_b, pool_v, sem_a, sem_b):
        wid = lax.axis_index("s") * NC + lax.axis_index("c")
        base = wid * BPW
        pltpu.sync_copy(x_hbm.at[pl.ds(base, BPW)], idx_v)

        bufs = (rows_a, rows_b)
        sems = (sem_a, sem_b)

        def fire(b, which):
            rows, sem = bufs[which], sems[which]
            pltpu.async_copy(tab_hbm.at[idx_v.at[b, 0]], rows.at[pl.ds(0, CHUNK)], sem)
            pltpu.async_copy(tab_hbm.at[idx_v.at[b, 1]], rows.at[pl.ds(CHUNK, CHUNK)], sem)

        def drain(b, which):
            rows, sem = bufs[which], sems[which]
            pltpu.make_async_copy(tab_hbm.at[idx_v.at[b, 0]], rows.at[pl.ds(0, CHUNK)], sem).wait()
            pltpu.make_async_copy(tab_hbm.at[idx_v.at[b, 1]], rows.at[pl.ds(CHUNK, CHUNK)], sem).wait()

        def accumulate(b, which):
            rows = bufs[which]

            def body(i, carry):
                a0e, a0o, a1e, a1o = carry
                r = i * UNROLL
                for u in range(UNROLL):
                    lo = rows[r + u, 0:16]
                    hi = rows[r + u, 16:32]
                    if u % 2 == 0:
                        a0e = a0e + lo
                        a1e = a1e + hi
                    else:
                        a0o = a0o + lo
                        a1o = a1o + hi
                return a0e, a0o, a1e, a1o

            z = jnp.zeros((16,), jnp.float32)
            a0e, a0o, a1e, a1o = lax.fori_loop(0, SEQ // UNROLL, body, (z, z, z, z))
            pool_v[b, 0:16] = a0e + a0o
            pool_v[b, 16:32] = a1e + a1o

        fire(0, 0)

        def step(i, _):
            b = 2 * i
            fire(b + 1, 1)
            drain(b, 0)
            accumulate(b, 0)

            @pl.when(i < BPW // 2 - 1)
            def _():
                fire(b + 2, 0)

            drain(b + 1, 1)
            accumulate(b + 1, 1)
            return 0

        lax.fori_loop(0, BPW // 2, step, 0)
        pltpu.sync_copy(pool_v, out_hbm.at[pl.ds(base, BPW)])

    return pool_k(xt3, emb_table)


def _mlp_tc(pooled_sum, age2, w1, wa, b1, w2, b2):
    def body(p_ref, age_ref, w1_ref, wa_ref, b1_ref, w2_ref, b2_ref, o_ref):
        pooled = p_ref[...] * jnp.float32(1.0 / SEQ)
        h = jnp.dot(pooled, w1_ref[...], preferred_element_type=jnp.float32)
        h = h + age_ref[...] * wa_ref[...] + b1_ref[...]
        logits = jnp.dot(h, w2_ref[...], preferred_element_type=jnp.float32)
        logits = logits + b2_ref[...]
        m = jnp.max(logits, axis=-1, keepdims=True)
        s = logits - m
        lse = jnp.log(jnp.sum(jnp.exp(s), axis=-1, keepdims=True))
        o_ref[...] = s - lse

    return pl.pallas_call(
        body,
        out_shape=jax.ShapeDtypeStruct((BATCH, OUT), jnp.float32),
    )(pooled_sum, age2, w1, wa, b1, w2, b2)


def kernel(x, age, emb_table, fc_w, fc_b, hid_w, hid_b):
    xt3 = jnp.transpose(x).astype(jnp.int32).reshape(BATCH, NCHUNK, CHUNK)
    pooled_sum = _pooled_sum_sc(xt3, emb_table)

    age2 = age.reshape(BATCH, 1)
    w1 = fc_w[:, :EMB].T
    wa = fc_w[:, EMB:].T
    b1 = fc_b.reshape(1, HIDDEN)
    w2 = hid_w.T
    b2 = hid_b.reshape(1, OUT)
    return _mlp_tc(pooled_sum, age2, w1, wa, b1, w2, b2)


# speedup vs baseline: 1.9663x; 1.0524x over previous
import functools

import jax
import jax.numpy as jnp
from jax import lax
from jax.experimental import pallas as pl
from jax.experimental.pallas import tpu as pltpu
from jax.experimental.pallas import tpu_sc as plsc

VOCAB = 1000000
EMB = 32
HIDDEN = 50
OUT = 100
SEQ = 200
BATCH = 4096

NC = 2
NS = 16
NW = NC * NS
BPW = BATCH // NW
NCHUNK = 2
CHUNK = SEQ // NCHUNK
UNROLL = 4
GRP = 4
NGRP = BPW // GRP


def _pooled_sum_sc(xt3, emb_table):
    mesh = plsc.VectorSubcoreMesh(core_axis_name="c", subcore_axis_name="s")

    @functools.partial(
        pl.kernel,
        mesh=mesh,
        compiler_params=pltpu.CompilerParams(use_tc_tiling_on_sc=False),
        out_type=jax.ShapeDtypeStruct((BATCH, EMB), jnp.float32),
        scratch_types=[
            pltpu.VMEM((BPW, NCHUNK, CHUNK), jnp.int32),
            pltpu.VMEM((GRP, NCHUNK, CHUNK, EMB), jnp.float32),
            pltpu.VMEM((GRP, NCHUNK, CHUNK, EMB), jnp.float32),
            pltpu.VMEM((BPW, EMB), jnp.float32),
            pltpu.SemaphoreType.DMA,
            pltpu.SemaphoreType.DMA,
        ],
    )
    def pool_k(x_hbm, tab_hbm, out_hbm, idx_v, rows_a, rows_b, pool_v, sem_a, sem_b):
        wid = lax.axis_index("s") * NC + lax.axis_index("c")
        base = wid * BPW
        pltpu.sync_copy(x_hbm.at[pl.ds(base, BPW)], idx_v)

        bufs = (rows_a, rows_b)
        sems = (sem_a, sem_b)

        def fire(g, which):
            rows, sem = bufs[which], sems[which]
            for gg in range(GRP):
                for c in range(NCHUNK):
                    pltpu.async_copy(
                        tab_hbm.at[idx_v.at[g * GRP + gg, c]], rows.at[gg, c], sem)

        def drain(g, which):
            rows, sem = bufs[which], sems[which]
            for gg in range(GRP):
                for c in range(NCHUNK):
                    pltpu.make_async_copy(
                        tab_hbm.at[idx_v.at[g * GRP + gg, c]], rows.at[gg, c], sem).wait()

        def accumulate(g, which):
            rows = bufs[which]
            for gg in range(GRP):

                def body(i, carry, gg=gg):
                    a0e, a0o, a1e, a1o = carry
                    r = i * UNROLL
                    for c in range(NCHUNK):
                        for u in range(UNROLL):
                            lo = rows[gg, c, r + u, 0:16]
                            hi = rows[gg, c, r + u, 16:32]
                            if u % 2 == 0:
                                a0e = a0e + lo
                                a1e = a1e + hi
                            else:
                                a0o = a0o + lo
                                a1o = a1o + hi
                    return a0e, a0o, a1e, a1o

                z = jnp.zeros((16,), jnp.float32)
                a0e, a0o, a1e, a1o = lax.fori_loop(0, CHUNK // UNROLL, body, (z, z, z, z))
                b = g * GRP + gg
                pool_v[b, 0:16] = a0e + a0o
                pool_v[b, 16:32] = a1e + a1o

        fire(0, 0)

        def step(i, _):
            g = 2 * i
            fire(g + 1, 1)
            drain(g, 0)
            accumulate(g, 0)

            @pl.when(i < NGRP // 2 - 1)
            def _():
                fire(g + 2, 0)

            drain(g + 1, 1)
            accumulate(g + 1, 1)
            return 0

        lax.fori_loop(0, NGRP // 2, step, 0)
        pltpu.sync_copy(pool_v, out_hbm.at[pl.ds(base, BPW)])

    return pool_k(xt3, emb_table)


def _mlp_tc(pooled_sum, age2, w1, wa, b1, w2, b2):
    def body(p_ref, age_ref, w1_ref, wa_ref, b1_ref, w2_ref, b2_ref, o_ref):
        pooled = p_ref[...] * jnp.float32(1.0 / SEQ)
        h = jnp.dot(pooled, w1_ref[...], preferred_element_type=jnp.float32)
        h = h + age_ref[...] * wa_ref[...] + b1_ref[...]
        logits = jnp.dot(h, w2_ref[...], preferred_element_type=jnp.float32)
        logits = logits + b2_ref[...]
        m = jnp.max(logits, axis=-1, keepdims=True)
        s = logits - m
        lse = jnp.log(jnp.sum(jnp.exp(s), axis=-1, keepdims=True))
        o_ref[...] = s - lse

    return pl.pallas_call(
        body,
        out_shape=jax.ShapeDtypeStruct((BATCH, OUT), jnp.float32),
    )(pooled_sum, age2, w1, wa, b1, w2, b2)


def kernel(x, age, emb_table, fc_w, fc_b, hid_w, hid_b):
    xt3 = jnp.transpose(x).astype(jnp.int32).reshape(BATCH, NCHUNK, CHUNK)
    pooled_sum = _pooled_sum_sc(xt3, emb_table)

    age2 = age.reshape(BATCH, 1)
    w1 = fc_w[:, :EMB].T
    wa = fc_w[:, EMB:].T
    b1 = fc_b.reshape(1, HIDDEN)
    w2 = hid_w.T
    b2 = hid_b.reshape(1, OUT)
    return _mlp_tc(pooled_sum, age2, w1, wa, b1, w2, b2)
